# Initial kernel scaffold; baseline (speedup 1.0000x reference)
import jax
import jax.numpy as jnp
from jax.experimental import pallas as pl


def kernel(features, boxes):
    raise NotImplementedError("write your pallas kernel here")


# speedup vs baseline: 11.1280x; 11.1280x over previous
import functools

import jax
import jax.numpy as jnp
from jax import lax
from jax.experimental import pallas as pl
from jax.experimental.pallas import tpu as pltpu
from jax.experimental.pallas import tpu_sc as plsc

H = W = 200
C = 128
N = 1024
OH, OW = 8, 32
S = OH * OW
HW_PAD = 40960
NWORKERS = 32
BOXES_PER_WORKER = N // NWORKERS
CHUNK = 128


def _transpose_body(f_ref, t_ref):
    t_ref[...] = f_ref[...].T


def _make_table(feat2d_padded):
    grid = HW_PAD // 512
    return pl.pallas_call(
        _transpose_body,
        grid=(grid,),
        in_specs=[pl.BlockSpec((C, 512), lambda i: (0, i))],
        out_specs=pl.BlockSpec((512, C), lambda i: (i, 0)),
        out_shape=jax.ShapeDtypeStruct((HW_PAD, C), jnp.float32),
    )(feat2d_padded)


def _coords_body(boxes_ref, idx_ref, w_ref):
    b = boxes_ref[...]
    left = b[:, 0:1]
    top = b[:, 1:2]
    right = b[:, 2:3]
    bottom = b[:, 3:4]
    width = right - left
    height = bottom - top
    ratio = (OW / OH) * height / width
    update = ratio > 1.0
    pad = width * (ratio - 1.0) * 0.5
    nl = jnp.where(update, left - pad, left)
    nr = jnp.where(update, right + pad, right)
    bw = (nr - nl) / OW
    bh = height / OH

    col = lax.broadcasted_iota(jnp.int32, (N, S), 1)
    ow = (col % OW).astype(jnp.float32)
    oh = (col // OW).astype(jnp.float32)
    xs = nl + (ow + 0.5) * bw
    ys = top + (oh + 0.5) * bh
    x0f = jnp.floor(xs)
    y0f = jnp.floor(ys)
    lx = xs - x0f
    ly = ys - y0f
    x0 = jnp.clip(x0f.astype(jnp.int32), 0, W - 1)
    x1 = jnp.minimum(x0 + 1, W - 1)
    y0 = jnp.clip(y0f.astype(jnp.int32), 0, H - 1)
    y1 = jnp.minimum(y0 + 1, H - 1)
    i00 = y0 * W + x0
    i01 = y0 * W + x1
    i10 = y1 * W + x0
    i11 = y1 * W + x1

    dp = (ratio - 1.0) / ratio * (OW / 2)
    keep = (ow >= dp) & (ow < (OW - dp))
    m = jnp.where(update & ~keep, 0.0, 1.0)
    w00 = (1.0 - ly) * (1.0 - lx) * m
    w01 = (1.0 - ly) * lx * m
    w10 = ly * (1.0 - lx) * m
    w11 = ly * lx * m

    for ch in range(2):
        sl = slice(ch * CHUNK, (ch + 1) * CHUNK)
        for k, arr in enumerate((i00, i01, i10, i11)):
            o = ch * 512 + k * CHUNK
            idx_ref[:, o:o + CHUNK] = arr[:, sl]
    for k, arr in enumerate((w00, w01, w10, w11)):
        w_ref[:, k * S:(k + 1) * S] = arr


def _make_coords(boxes):
    return pl.pallas_call(
        _coords_body,
        out_shape=(
            jax.ShapeDtypeStruct((N, 2 * 4 * CHUNK), jnp.int32),
            jax.ShapeDtypeStruct((N, 4 * S), jnp.float32),
        ),
    )(boxes)


def _sc_body(idx_hbm, w_hbm, table_hbm, out_hbm,
             w_v, idx_v, g00, g01, g10, g11, obuf, sem):
    wid = lax.axis_index("s") * 2 + lax.axis_index("c")
    iota = lax.broadcasted_iota(jnp.int32, (16,), 0)
    scat_base = iota * S

    @pl.loop(0, BOXES_PER_WORKER)
    def _box(k):
        b = wid * BOXES_PER_WORKER + k
        pltpu.sync_copy(w_hbm.at[b], w_v)

        @pl.loop(0, S // CHUNK)
        def _chunk(ch):
            pltpu.sync_copy(idx_hbm.at[b, ch], idx_v)
            cps = [
                pltpu.async_copy(table_hbm.at[idx_v.at[k2]], g, sem)
                for k2, g in enumerate((g00, g01, g10, g11))
            ]
            for cp in cps:
                cp.wait()

            @pl.loop(0, CHUNK)
            def _samp(sl):
                s = ch * CHUNK + sl
                w00v = plsc.load_gather(w_v, [jnp.full((16,), 0 * S, jnp.int32) + s])
                w01v = plsc.load_gather(w_v, [jnp.full((16,), 1 * S, jnp.int32) + s])
                w10v = plsc.load_gather(w_v, [jnp.full((16,), 2 * S, jnp.int32) + s])
                w11v = plsc.load_gather(w_v, [jnp.full((16,), 3 * S, jnp.int32) + s])
                for kb in range(C // 16):
                    o = pl.ds(kb * 16, 16)
                    a00 = g00[sl, o]
                    a01 = g01[sl, o]
                    a10 = g10[sl, o]
                    a11 = g11[sl, o]
                    acc = (w00v * a00 + w01v * a01) + (w10v * a10 + w11v * a11)
                    plsc.store_scatter(obuf, [scat_base + (kb * 16 * S + s)], acc)

        pltpu.sync_copy(obuf, out_hbm.at[b])


def _sc_pool(idx, w, table):
    mesh = plsc.VectorSubcoreMesh(core_axis_name="c", subcore_axis_name="s")
    return pl.kernel(
        _sc_body,
        out_type=jax.ShapeDtypeStruct((N, C * S), jnp.float32),
        mesh=mesh,
        compiler_params=pltpu.CompilerParams(needs_layout_passes=False),
        scratch_types=[
            pltpu.VMEM((4 * S,), jnp.float32),
            pltpu.VMEM((4, CHUNK), jnp.int32),
            pltpu.VMEM((CHUNK, C), jnp.float32),
            pltpu.VMEM((CHUNK, C), jnp.float32),
            pltpu.VMEM((CHUNK, C), jnp.float32),
            pltpu.VMEM((CHUNK, C), jnp.float32),
            pltpu.VMEM((C * S,), jnp.float32),
            pltpu.SemaphoreType.DMA,
        ],
    )(idx, w, table)


def kernel(features, boxes):
    feat2d = features.reshape(C, H * W)
    feat2d = jnp.pad(feat2d, ((0, 0), (0, HW_PAD - H * W)))
    table = _make_table(feat2d)
    idx, w = _make_coords(boxes)
    idx = idx.reshape(N, 2, 4, CHUNK)
    out = _sc_pool(idx, w, table)
    return out.reshape(N, C, OH, OW)


# speedup vs baseline: 12.8983x; 1.1591x over previous
import functools

import jax
import jax.numpy as jnp
from jax import lax
from jax.experimental import pallas as pl
from jax.experimental.pallas import tpu as pltpu
from jax.experimental.pallas import tpu_sc as plsc

H = W = 200
C = 128
N = 1024
OH, OW = 8, 32
S = OH * OW
HW_PAD = 40960
NWORKERS = 32
BOXES_PER_WORKER = N // NWORKERS
CHUNK = 64
NCH = S // CHUNK


def _transpose_body(f_ref, t_ref):
    t_ref[...] = f_ref[...].T


def _make_table(feat2d_padded):
    grid = HW_PAD // 512
    return pl.pallas_call(
        _transpose_body,
        grid=(grid,),
        in_specs=[pl.BlockSpec((C, 512), lambda i: (0, i))],
        out_specs=pl.BlockSpec((512, C), lambda i: (i, 0)),
        out_shape=jax.ShapeDtypeStruct((HW_PAD, C), jnp.float32),
    )(feat2d_padded)


def _coords_body(boxes_ref, idx_ref, w_ref):
    b = boxes_ref[...]
    left = b[:, 0:1]
    top = b[:, 1:2]
    right = b[:, 2:3]
    bottom = b[:, 3:4]
    width = right - left
    height = bottom - top
    ratio = (OW / OH) * height / width
    update = ratio > 1.0
    pad = width * (ratio - 1.0) * 0.5
    nl = jnp.where(update, left - pad, left)
    nr = jnp.where(update, right + pad, right)
    bw = (nr - nl) / OW
    bh = height / OH

    col = lax.broadcasted_iota(jnp.int32, (N, S), 1)
    ow = (col % OW).astype(jnp.float32)
    oh = (col // OW).astype(jnp.float32)
    xs = nl + (ow + 0.5) * bw
    ys = top + (oh + 0.5) * bh
    x0f = jnp.floor(xs)
    y0f = jnp.floor(ys)
    lx = xs - x0f
    ly = ys - y0f
    x0 = jnp.clip(x0f.astype(jnp.int32), 0, W - 1)
    x1 = jnp.minimum(x0 + 1, W - 1)
    y0 = jnp.clip(y0f.astype(jnp.int32), 0, H - 1)
    y1 = jnp.minimum(y0 + 1, H - 1)
    i00 = y0 * W + x0
    i01 = y0 * W + x1
    i10 = y1 * W + x0
    i11 = y1 * W + x1

    dp = (ratio - 1.0) / ratio * (OW / 2)
    keep = (ow >= dp) & (ow < (OW - dp))
    m = jnp.where(update & ~keep, 0.0, 1.0)
    w00 = (1.0 - ly) * (1.0 - lx) * m
    w01 = (1.0 - ly) * lx * m
    w10 = ly * (1.0 - lx) * m
    w11 = ly * lx * m

    for ch in range(NCH):
        sl = slice(ch * CHUNK, (ch + 1) * CHUNK)
        for k, arr in enumerate((i00, i01, i10, i11)):
            o = (ch * 4 + k) * CHUNK
            idx_ref[:, o:o + CHUNK] = arr[:, sl]
    for k, arr in enumerate((w00, w01, w10, w11)):
        w_ref[:, k * S:(k + 1) * S] = arr


def _make_coords(boxes):
    return pl.pallas_call(
        _coords_body,
        out_shape=(
            jax.ShapeDtypeStruct((N, NCH * 4 * CHUNK), jnp.int32),
            jax.ShapeDtypeStruct((N, 4 * S), jnp.float32),
        ),
    )(boxes)


def _sc_body(idx_hbm, w_hbm, table_hbm, out_hbm,
             w_v, idx_v, ga0, ga1, ga2, ga3, gb0, gb1, gb2, gb3, obuf,
             gsem_a, gsem_b, wsem, osem):
    wid = lax.axis_index("s") * 2 + lax.axis_index("c")
    iota = lax.broadcasted_iota(jnp.int32, (16,), 0)
    scat_base = iota * S
    gsets = ((ga0, ga1, ga2, ga3, gsem_a), (gb0, gb1, gb2, gb3, gsem_b))
    b0 = wid * BOXES_PER_WORKER

    pltpu.sync_copy(w_hbm.at[b0], w_v.at[pl.ds(0, 4 * S)])
    pltpu.sync_copy(idx_hbm.at[b0], idx_v.at[0])

    @pl.loop(0, BOXES_PER_WORKER)
    def _box(k):
        b = b0 + k
        nb = lax.rem(k, 2)
        nbn = lax.rem(k + 1, 2)

        def issue(ch):
            gset = gsets[ch % 2]
            return [
                pltpu.async_copy(table_hbm.at[idx_v.at[nb, ch, c2]],
                                 gset[c2], gset[4])
                for c2 in range(4)
            ]

        cps_prev = issue(0)
        bn = jnp.minimum(b + 1, N - 1)
        pltpu.async_copy(w_hbm.at[bn], w_v.at[pl.ds(nbn * 4 * S, 4 * S)], wsem)
        pltpu.async_copy(idx_hbm.at[bn], idx_v.at[nbn], wsem)

        @pl.when(k > 0)
        def _():
            pltpu.make_async_copy(w_hbm.at[b], w_v.at[pl.ds(0, 4 * S)], wsem).wait()
            pltpu.make_async_copy(idx_hbm.at[b], idx_v.at[0], wsem).wait()
            pltpu.make_async_copy(out_hbm.at[b], obuf, osem).wait()

        for ch in range(NCH):
            cps_next = issue(ch + 1) if ch + 1 < NCH else None
            for cp in cps_prev:
                cp.wait()
            cps_prev = cps_next
            gset = gsets[ch % 2]
            g0, g1, g2, g3 = gset[0], gset[1], gset[2], gset[3]

            @pl.loop(0, CHUNK, unroll=2)
            def _samp(sl):
                s = ch * CHUNK + sl
                wb = nb * (4 * S) + s
                w00v = plsc.load_gather(w_v, [jnp.full((16,), 0 * S, jnp.int32) + wb])
                w01v = plsc.load_gather(w_v, [jnp.full((16,), 1 * S, jnp.int32) + wb])
                w10v = plsc.load_gather(w_v, [jnp.full((16,), 2 * S, jnp.int32) + wb])
                w11v = plsc.load_gather(w_v, [jnp.full((16,), 3 * S, jnp.int32) + wb])
                for kb in range(C // 16):
                    o = pl.ds(kb * 16, 16)
                    a00 = g0[sl, o]
                    a01 = g1[sl, o]
                    a10 = g2[sl, o]
                    a11 = g3[sl, o]
                    acc = (w00v * a00 + w01v * a01) + (w10v * a10 + w11v * a11)
                    plsc.store_scatter(obuf, [scat_base + (kb * 16 * S + s)], acc)

        pltpu.async_copy(obuf, out_hbm.at[b], osem)

    pltpu.make_async_copy(out_hbm.at[b0], obuf, osem).wait()
    pltpu.make_async_copy(w_hbm.at[b0], w_v.at[pl.ds(0, 4 * S)], wsem).wait()
    pltpu.make_async_copy(idx_hbm.at[b0], idx_v.at[0], wsem).wait()


def _sc_pool(idx, w, table):
    mesh = plsc.VectorSubcoreMesh(core_axis_name="c", subcore_axis_name="s")
    return pl.kernel(
        _sc_body,
        out_type=jax.ShapeDtypeStruct((N, C * S), jnp.float32),
        mesh=mesh,
        compiler_params=pltpu.CompilerParams(needs_layout_passes=False),
        scratch_types=[
            pltpu.VMEM((2 * 4 * S,), jnp.float32),
            pltpu.VMEM((2, NCH, 4, CHUNK), jnp.int32),
            pltpu.VMEM((CHUNK, C), jnp.float32),
            pltpu.VMEM((CHUNK, C), jnp.float32),
            pltpu.VMEM((CHUNK, C), jnp.float32),
            pltpu.VMEM((CHUNK, C), jnp.float32),
            pltpu.VMEM((CHUNK, C), jnp.float32),
            pltpu.VMEM((CHUNK, C), jnp.float32),
            pltpu.VMEM((CHUNK, C), jnp.float32),
            pltpu.VMEM((CHUNK, C), jnp.float32),
            pltpu.VMEM((C * S,), jnp.float32),
            pltpu.SemaphoreType.DMA,
            pltpu.SemaphoreType.DMA,
            pltpu.SemaphoreType.DMA,
            pltpu.SemaphoreType.DMA,
        ],
    )(idx, w, table)


def kernel(features, boxes):
    feat2d = features.reshape(C, H * W)
    feat2d = jnp.pad(feat2d, ((0, 0), (0, HW_PAD - H * W)))
    table = _make_table(feat2d)
    idx, w = _make_coords(boxes)
    idx = idx.reshape(N, NCH, 4, CHUNK)
    out = _sc_pool(idx, w, table)
    return out.reshape(N, C, OH, OW)


# speedup vs baseline: 18.0052x; 1.3959x over previous
import functools

import jax
import jax.numpy as jnp
from jax import lax
from jax.experimental import pallas as pl
from jax.experimental.pallas import tpu as pltpu
from jax.experimental.pallas import tpu_sc as plsc

H = W = 200
C = 128
N = 1024
OH, OW = 8, 32
S = OH * OW
HW_PAD = 40960
NWORKERS = 32
BOXES_PER_WORKER = N // NWORKERS
CHUNK = 64
NCH = S // CHUNK


def _transpose_body(f_ref, t_ref):
    t_ref[...] = f_ref[...].T


def _make_table(feat2d_padded):
    grid = HW_PAD // 512
    return pl.pallas_call(
        _transpose_body,
        grid=(grid,),
        in_specs=[pl.BlockSpec((C, 512), lambda i: (0, i))],
        out_specs=pl.BlockSpec((512, C), lambda i: (i, 0)),
        out_shape=jax.ShapeDtypeStruct((HW_PAD, C), jnp.float32),
    )(feat2d_padded)


def _coords_body(boxes_ref, idx_ref, w_ref):
    b = boxes_ref[...]
    left = b[:, 0:1]
    top = b[:, 1:2]
    right = b[:, 2:3]
    bottom = b[:, 3:4]
    width = right - left
    height = bottom - top
    ratio = (OW / OH) * height / width
    update = ratio > 1.0
    pad = width * (ratio - 1.0) * 0.5
    nl = jnp.where(update, left - pad, left)
    nr = jnp.where(update, right + pad, right)
    bw = (nr - nl) / OW
    bh = height / OH

    col = lax.broadcasted_iota(jnp.int32, (N, S), 1)
    ow = (col % OW).astype(jnp.float32)
    oh = (col // OW).astype(jnp.float32)
    xs = nl + (ow + 0.5) * bw
    ys = top + (oh + 0.5) * bh
    x0f = jnp.floor(xs)
    y0f = jnp.floor(ys)
    lx = xs - x0f
    ly = ys - y0f
    x0 = jnp.clip(x0f.astype(jnp.int32), 0, W - 1)
    x1 = jnp.minimum(x0 + 1, W - 1)
    y0 = jnp.clip(y0f.astype(jnp.int32), 0, H - 1)
    y1 = jnp.minimum(y0 + 1, H - 1)
    i00 = y0 * W + x0
    i01 = y0 * W + x1
    i10 = y1 * W + x0
    i11 = y1 * W + x1

    dp = (ratio - 1.0) / ratio * (OW / 2)
    keep = (ow >= dp) & (ow < (OW - dp))
    m = jnp.where(update & ~keep, 0.0, 1.0)
    w00 = (1.0 - ly) * (1.0 - lx) * m
    w01 = (1.0 - ly) * lx * m
    w10 = ly * (1.0 - lx) * m
    w11 = ly * lx * m

    for ch in range(NCH):
        sl = slice(ch * CHUNK, (ch + 1) * CHUNK)
        for k, arr in enumerate((i00, i01, i10, i11)):
            o = (ch * 4 + k) * CHUNK
            idx_ref[:, o:o + CHUNK] = arr[:, sl]
    for k, arr in enumerate((w00, w01, w10, w11)):
        w_ref[:, k * S:(k + 1) * S] = arr


def _make_coords(boxes):
    return pl.pallas_call(
        _coords_body,
        out_shape=(
            jax.ShapeDtypeStruct((N, NCH * 4 * CHUNK), jnp.int32),
            jax.ShapeDtypeStruct((N, 4 * S), jnp.float32),
        ),
    )(boxes)


def _sc_body(idx_hbm, w_hbm, table_hbm, out_hbm,
             w_v, idx_v, ga0, ga1, ga2, ga3, gb0, gb1, gb2, gb3, obuf,
             gsem_a, gsem_b, wsem, osem):
    wid = lax.axis_index("s") * 2 + lax.axis_index("c")
    iota = lax.broadcasted_iota(jnp.int32, (16,), 0)
    scat_base = iota * S
    gsets = ((ga0, ga1, ga2, ga3, gsem_a), (gb0, gb1, gb2, gb3, gsem_b))
    b0 = wid * BOXES_PER_WORKER

    pltpu.sync_copy(w_hbm.at[b0], w_v.at[pl.ds(0, 4 * S)])
    pltpu.sync_copy(idx_hbm.at[b0], idx_v.at[0])

    @pl.loop(0, BOXES_PER_WORKER)
    def _box(k):
        b = b0 + k
        nb = lax.rem(k, 2)
        nbn = lax.rem(k + 1, 2)

        def issue(ch):
            gset = gsets[ch % 2]
            return [
                pltpu.async_copy(table_hbm.at[idx_v.at[nb, ch, c2]],
                                 gset[c2], gset[4])
                for c2 in range(4)
            ]

        cps_prev = issue(0)
        bn = jnp.minimum(b + 1, N - 1)
        pltpu.async_copy(w_hbm.at[bn], w_v.at[pl.ds(nbn * 4 * S, 4 * S)], wsem)
        pltpu.async_copy(idx_hbm.at[bn], idx_v.at[nbn], wsem)

        @pl.when(k > 0)
        def _():
            pltpu.make_async_copy(w_hbm.at[b], w_v.at[pl.ds(0, 4 * S)], wsem).wait()
            pltpu.make_async_copy(idx_hbm.at[b], idx_v.at[0], wsem).wait()
            pltpu.make_async_copy(out_hbm.at[b], obuf, osem).wait()

        for ch in range(NCH):
            cps_next = issue(ch + 1) if ch + 1 < NCH else None
            for cp in cps_prev:
                cp.wait()
            cps_prev = cps_next
            gset = gsets[ch % 2]
            g0, g1, g2, g3 = gset[0], gset[1], gset[2], gset[3]

            @plsc.parallel_loop(0, CHUNK, unroll=4)
            def _samp(sl):
                s = ch * CHUNK + sl
                wb = nb * (4 * S) + s
                w00v = plsc.load_gather(w_v, [jnp.full((16,), 0 * S, jnp.int32) + wb])
                w01v = plsc.load_gather(w_v, [jnp.full((16,), 1 * S, jnp.int32) + wb])
                w10v = plsc.load_gather(w_v, [jnp.full((16,), 2 * S, jnp.int32) + wb])
                w11v = plsc.load_gather(w_v, [jnp.full((16,), 3 * S, jnp.int32) + wb])
                for kb in range(C // 16):
                    o = pl.ds(kb * 16, 16)
                    a00 = g0[sl, o]
                    a01 = g1[sl, o]
                    a10 = g2[sl, o]
                    a11 = g3[sl, o]
                    acc = (w00v * a00 + w01v * a01) + (w10v * a10 + w11v * a11)
                    plsc.store_scatter(obuf, [scat_base + (kb * 16 * S + s)], acc)

        pltpu.async_copy(obuf, out_hbm.at[b], osem)

    pltpu.make_async_copy(out_hbm.at[b0], obuf, osem).wait()
    pltpu.make_async_copy(w_hbm.at[b0], w_v.at[pl.ds(0, 4 * S)], wsem).wait()
    pltpu.make_async_copy(idx_hbm.at[b0], idx_v.at[0], wsem).wait()


def _sc_pool(idx, w, table):
    mesh = plsc.VectorSubcoreMesh(core_axis_name="c", subcore_axis_name="s")
    return pl.kernel(
        _sc_body,
        out_type=jax.ShapeDtypeStruct((N, C * S), jnp.float32),
        mesh=mesh,
        compiler_params=pltpu.CompilerParams(needs_layout_passes=False),
        scratch_types=[
            pltpu.VMEM((2 * 4 * S,), jnp.float32),
            pltpu.VMEM((2, NCH, 4, CHUNK), jnp.int32),
            pltpu.VMEM((CHUNK, C), jnp.float32),
            pltpu.VMEM((CHUNK, C), jnp.float32),
            pltpu.VMEM((CHUNK, C), jnp.float32),
            pltpu.VMEM((CHUNK, C), jnp.float32),
            pltpu.VMEM((CHUNK, C), jnp.float32),
            pltpu.VMEM((CHUNK, C), jnp.float32),
            pltpu.VMEM((CHUNK, C), jnp.float32),
            pltpu.VMEM((CHUNK, C), jnp.float32),
            pltpu.VMEM((C * S,), jnp.float32),
            pltpu.SemaphoreType.DMA,
            pltpu.SemaphoreType.DMA,
            pltpu.SemaphoreType.DMA,
            pltpu.SemaphoreType.DMA,
        ],
    )(idx, w, table)


def kernel(features, boxes):
    feat2d = features.reshape(C, H * W)
    feat2d = jnp.pad(feat2d, ((0, 0), (0, HW_PAD - H * W)))
    table = _make_table(feat2d)
    idx, w = _make_coords(boxes)
    idx = idx.reshape(N, NCH, 4, CHUNK)
    out = _sc_pool(idx, w, table)
    return out.reshape(N, C, OH, OW)


# speedup vs baseline: 46.0500x; 2.5576x over previous
import functools

import jax
import jax.numpy as jnp
from jax import lax
from jax.experimental import pallas as pl
from jax.experimental.pallas import tpu as pltpu
from jax.experimental.pallas import tpu_sc as plsc

H = W = 200
C = 128
N = 1024
OH, OW = 8, 32
S = OH * OW
NWORKERS = 32
BOXES_PER_WORKER = N // NWORKERS
CHUNK = 64
NCH = S // CHUNK


def _coords_body(boxes_ref, idx_ref, w_ref):
    b = boxes_ref[...]
    left = b[:, 0:1]
    top = b[:, 1:2]
    right = b[:, 2:3]
    bottom = b[:, 3:4]
    width = right - left
    height = bottom - top
    ratio = (OW / OH) * height / width
    update = ratio > 1.0
    pad = width * (ratio - 1.0) * 0.5
    nl = jnp.where(update, left - pad, left)
    nr = jnp.where(update, right + pad, right)
    bw = (nr - nl) / OW
    bh = height / OH

    col = lax.broadcasted_iota(jnp.int32, (N, S), 1)
    ow = (col % OW).astype(jnp.float32)
    oh = (col // OW).astype(jnp.float32)
    xs = nl + (ow + 0.5) * bw
    ys = top + (oh + 0.5) * bh
    x0f = jnp.floor(xs)
    y0f = jnp.floor(ys)
    lx = xs - x0f
    ly = ys - y0f
    x0 = jnp.clip(x0f.astype(jnp.int32), 0, W - 1)
    x1 = jnp.minimum(x0 + 1, W - 1)
    y0 = jnp.clip(y0f.astype(jnp.int32), 0, H - 1)
    y1 = jnp.minimum(y0 + 1, H - 1)
    i00 = y0 * W + x0
    i01 = y0 * W + x1
    i10 = y1 * W + x0
    i11 = y1 * W + x1

    dp = (ratio - 1.0) / ratio * (OW / 2)
    keep = (ow >= dp) & (ow < (OW - dp))
    m = jnp.where(update & ~keep, 0.0, 1.0)
    w00 = (1.0 - ly) * (1.0 - lx) * m
    w01 = (1.0 - ly) * lx * m
    w10 = ly * (1.0 - lx) * m
    w11 = ly * lx * m

    for j in range(8):
        corner = j // 2
        half = slice((j % 2) * 128, (j % 2 + 1) * 128)
        idx_ref[:, j, :] = (i00, i01, i10, i11)[corner][:, half]
        w_ref[:, j, :] = (w00, w01, w10, w11)[corner][:, half]


def _make_coords(boxes):
    return pl.pallas_call(
        _coords_body,
        out_shape=(
            jax.ShapeDtypeStruct((N, 8, 128), jnp.int32),
            jax.ShapeDtypeStruct((N, 8, 128), jnp.float32),
        ),
    )(boxes)


def _sc_body(idx_hbm, w_hbm, table_hbm, out_hbm,
             w_v, idx_v, ga0, ga1, ga2, ga3, gb0, gb1, gb2, gb3, obuf,
             gsem_a, gsem_b, wsem, osem):
    wid = lax.axis_index("s") * 2 + lax.axis_index("c")
    gsets = ((ga0, ga1, ga2, ga3, gsem_a), (gb0, gb1, gb2, gb3, gsem_b))
    b0 = wid * BOXES_PER_WORKER

    for j in range(8):
        pltpu.sync_copy(w_hbm.at[b0, j], w_v.at[pl.ds(j * 128, 128)])
    pltpu.sync_copy(idx_hbm.at[b0], idx_v.at[0])

    @pl.loop(0, BOXES_PER_WORKER)
    def _box(k):
        b = b0 + k
        nb = lax.rem(k, 2)
        nbn = lax.rem(k + 1, 2)

        def issue(ch):
            gset = gsets[ch % 2]
            return [
                pltpu.async_copy(
                    table_hbm.at[idx_v.at[nb, c2 * 2 + ch // 2,
                                          pl.ds((ch % 2) * CHUNK, CHUNK)]],
                    gset[c2], gset[4])
                for c2 in range(4)
            ]

        cps_prev = issue(0)
        bn = jnp.minimum(b + 1, N - 1)
        for j in range(8):
            pltpu.async_copy(w_hbm.at[bn, j],
                             w_v.at[pl.ds(nbn * 1024 + j * 128, 128)], wsem)
        pltpu.async_copy(idx_hbm.at[bn], idx_v.at[nbn], wsem)

        @pl.when(k > 0)
        def _():
            for j in range(8):
                pltpu.make_async_copy(w_hbm.at[b, j],
                                      w_v.at[pl.ds(j * 128, 128)], wsem).wait()
            pltpu.make_async_copy(idx_hbm.at[b], idx_v.at[0], wsem).wait()

        for ch in range(NCH):
            cps_next = issue(ch + 1) if ch + 1 < NCH else None
            for cp in cps_prev:
                cp.wait()
            cps_prev = cps_next
            gset = gsets[ch % 2]
            g0, g1, g2, g3 = gset[0], gset[1], gset[2], gset[3]
            ob = ch % 2

            if ch < 2:
                @pl.when(k > 0)
                def _():
                    pltpu.make_async_copy(out_hbm.at[b, pl.ds(0, CHUNK)],
                                          obuf.at[ob], osem).wait()
            else:
                pltpu.make_async_copy(out_hbm.at[b, pl.ds(0, CHUNK)],
                                      obuf.at[ob], osem).wait()

            @plsc.parallel_loop(0, CHUNK, unroll=4)
            def _samp(sl):
                s = ch * CHUNK + sl
                wb = nb * 1024 + s
                w00v = plsc.load_gather(w_v, [jnp.full((16,), 0 * S, jnp.int32) + wb])
                w01v = plsc.load_gather(w_v, [jnp.full((16,), 1 * S, jnp.int32) + wb])
                w10v = plsc.load_gather(w_v, [jnp.full((16,), 2 * S, jnp.int32) + wb])
                w11v = plsc.load_gather(w_v, [jnp.full((16,), 3 * S, jnp.int32) + wb])
                for kb in range(C // 16):
                    o = pl.ds(kb * 16, 16)
                    a00 = g0[sl, o]
                    a01 = g1[sl, o]
                    a10 = g2[sl, o]
                    a11 = g3[sl, o]
                    acc = (w00v * a00 + w01v * a01) + (w10v * a10 + w11v * a11)
                    obuf[ob, sl, o] = acc

            pltpu.async_copy(obuf.at[ob],
                             out_hbm.at[b, pl.ds(ch * CHUNK, CHUNK)], osem)

    pltpu.make_async_copy(out_hbm.at[b0, pl.ds(0, CHUNK)], obuf.at[0], osem).wait()
    pltpu.make_async_copy(out_hbm.at[b0, pl.ds(0, CHUNK)], obuf.at[1], osem).wait()
    for j in range(8):
        pltpu.make_async_copy(w_hbm.at[b0, j],
                              w_v.at[pl.ds(j * 128, 128)], wsem).wait()
    pltpu.make_async_copy(idx_hbm.at[b0], idx_v.at[0], wsem).wait()


def _sc_pool(idx, w, table):
    mesh = plsc.VectorSubcoreMesh(core_axis_name="c", subcore_axis_name="s")
    return pl.kernel(
        _sc_body,
        out_type=jax.ShapeDtypeStruct((N, S, C), jnp.float32),
        mesh=mesh,
        compiler_params=pltpu.CompilerParams(needs_layout_passes=False),
        scratch_types=[
            pltpu.VMEM((2 * 1024,), jnp.float32),
            pltpu.VMEM((2, 8, 128), jnp.int32),
            pltpu.VMEM((CHUNK, C), jnp.float32),
            pltpu.VMEM((CHUNK, C), jnp.float32),
            pltpu.VMEM((CHUNK, C), jnp.float32),
            pltpu.VMEM((CHUNK, C), jnp.float32),
            pltpu.VMEM((CHUNK, C), jnp.float32),
            pltpu.VMEM((CHUNK, C), jnp.float32),
            pltpu.VMEM((CHUNK, C), jnp.float32),
            pltpu.VMEM((CHUNK, C), jnp.float32),
            pltpu.VMEM((2, CHUNK, C), jnp.float32),
            pltpu.SemaphoreType.DMA,
            pltpu.SemaphoreType.DMA,
            pltpu.SemaphoreType.DMA,
            pltpu.SemaphoreType.DMA,
        ],
    )(idx, w, table)


def kernel(features, boxes):
    table = features[0].transpose(1, 2, 0).reshape(H * W, C)
    idx, w = _make_coords(boxes)
    out = _sc_pool(idx, w, table)
    return out.reshape(N, OH, OW, C).transpose(0, 3, 1, 2)
